# baseline (speedup 1.0000x reference)
import jax
import jax.numpy as jnp
from jax.experimental import pallas as pl

NUM_CLASSES = 10
NUM_YAW = 2
BOX_DOF = 7
C_IN = 384
B, NY, NX = 4, 200, 176
HW = NY * NX
TILE = 1408


def _head_kernel(x_ref, wc_ref, bc_ref, wr_ref, br_ref, cls_ref, reg_ref):
    x = x_ref[0]
    cls_ref[0] = (
        jnp.dot(wc_ref[...], x, preferred_element_type=jnp.float32) + bc_ref[...]
    )
    reg_ref[0] = (
        jnp.dot(wr_ref[...], x, preferred_element_type=jnp.float32) + br_ref[...]
    )


def kernel(feature_map, W_cls, b_cls, W_reg, b_reg):
    c_cls = NUM_CLASSES * NUM_YAW
    c_reg = NUM_CLASSES * NUM_YAW * BOX_DOF
    x = feature_map.reshape(B, C_IN, HW)
    bc = b_cls.reshape(c_cls, 1)
    br = b_reg.reshape(c_reg, 1)

    nt = HW // TILE
    cls_out, reg_out = pl.pallas_call(
        _head_kernel,
        grid=(B, nt),
        in_specs=[
            pl.BlockSpec((1, C_IN, TILE), lambda b, t: (b, 0, t)),
            pl.BlockSpec((c_cls, C_IN), lambda b, t: (0, 0)),
            pl.BlockSpec((c_cls, 1), lambda b, t: (0, 0)),
            pl.BlockSpec((c_reg, C_IN), lambda b, t: (0, 0)),
            pl.BlockSpec((c_reg, 1), lambda b, t: (0, 0)),
        ],
        out_specs=[
            pl.BlockSpec((1, c_cls, TILE), lambda b, t: (b, 0, t)),
            pl.BlockSpec((1, c_reg, TILE), lambda b, t: (b, 0, t)),
        ],
        out_shape=[
            jax.ShapeDtypeStruct((B, c_cls, HW), jnp.float32),
            jax.ShapeDtypeStruct((B, c_reg, HW), jnp.float32),
        ],
    )(x, W_cls, bc, W_reg, br)

    cls_map = cls_out.reshape(B, NUM_CLASSES, NUM_YAW, NY, NX)
    reg_map = reg_out.reshape(B, NUM_CLASSES, BOX_DOF, NUM_YAW, NY, NX)
    reg_map = jnp.transpose(reg_map, (0, 1, 3, 4, 5, 2))
    return (cls_map, reg_map)
